# Initial kernel scaffold; baseline (speedup 1.0000x reference)
import jax
import jax.numpy as jnp
from jax.experimental import pallas as pl


def kernel(input_ids, word_embeddings, position_embeddings, token_type_embeddings, ln_gamma, ln_beta):
    raise NotImplementedError("write your pallas kernel here")


# speedup vs baseline: 2.9054x; 2.9054x over previous
import functools

import jax
import jax.numpy as jnp
from jax import lax
from jax.experimental import pallas as pl
from jax.experimental.pallas import tpu as pltpu
from jax.experimental.pallas import tpu_sc as plsc

NC = 2
NS = 16
NW = NC * NS
L = 16

VOCAB = 100000
HID = 128
B = 1024
S = 200
HC = HID // L
K = 40
TOK_PER_W = B * S // NW
CH = TOK_PER_W // K
EPS = 1e-12

_MESH = plsc.VectorSubcoreMesh(
    core_axis_name="c", subcore_axis_name="s", num_cores=NC, num_subcores=NS
)


@functools.partial(
    pl.kernel,
    mesh=_MESH,
    compiler_params=pltpu.CompilerParams(needs_layout_passes=False),
    out_type=jax.ShapeDtypeStruct((B * S, HID), jnp.float32),
    scratch_types=[
        pltpu.VMEM((CH, K), jnp.int32),
        pltpu.VMEM((K, HID), jnp.float32),
        pltpu.VMEM((S, HID), jnp.float32),
        pltpu.VMEM((HID,), jnp.float32),
        pltpu.VMEM((HID,), jnp.float32),
        pltpu.VMEM((HID,), jnp.float32),
        pltpu.SemaphoreType.DMA,
    ],
)
def _embed_ln(ids_hbm, pe_hbm, tte_hbm, gam_hbm, bet_hbm, table_hbm, out_hbm,
              idx_v, rows_v, bias_v, tte_v, gam_v, bet_v, sem):
    wid = lax.axis_index("s") * NC + lax.axis_index("c")

    pltpu.sync_copy(pe_hbm, bias_v)
    pltpu.sync_copy(tte_hbm, tte_v)
    pltpu.sync_copy(gam_hbm, gam_v)
    pltpu.sync_copy(bet_hbm, bet_v)
    pltpu.sync_copy(ids_hbm.at[wid], idx_v)

    ttec = [tte_v[pl.ds(c * L, L)] for c in range(HC)]

    def bias_body(t, carry):
        for c in range(HC):
            bias_v[t, pl.ds(c * L, L)] = bias_v[t, pl.ds(c * L, L)] + ttec[c]
        return carry

    lax.fori_loop(0, S, bias_body, 0)

    gam = [gam_v[pl.ds(c * L, L)] for c in range(HC)]
    bet = [bet_v[pl.ds(c * L, L)] for c in range(HC)]
    base = wid * TOK_PER_W

    lane = lax.iota(jnp.int32, L)
    dnums = lax.GatherDimensionNumbers(
        offset_dims=(), collapsed_slice_dims=(0,), start_index_map=(0,))

    def xlane_sum(v):
        for sh in (8, 4, 2, 1):
            perm = lax.bitwise_xor(lane, jnp.int32(sh))
            swapped = lax.gather(
                v, perm[:, None], dnums, slice_sizes=(1,),
                mode=lax.GatherScatterMode.PROMISE_IN_BOUNDS)
            v = v + swapped
        return v

    def chunk_body(j, carry):
        pltpu.async_copy(table_hbm.at[idx_v.at[j]], rows_v, sem).wait()
        poff = (j % (S // K)) * K

        def tok_body(t, tcarry):
            xs = []
            for c in range(HC):
                xs.append(rows_v[t, pl.ds(c * L, L)]
                          + bias_v[poff + t, pl.ds(c * L, L)])
            s1 = xs[0]
            s2 = xs[0] * xs[0]
            for c in range(1, HC):
                s1 = s1 + xs[c]
                s2 = s2 + xs[c] * xs[c]
            tot = xlane_sum(s1)
            tot2 = xlane_sum(s2)
            mean = tot * (1.0 / HID)
            var = tot2 * (1.0 / HID) - mean * mean + EPS
            i = plsc.bitcast(var, jnp.int32)
            i = jnp.int32(0x5F3759DF) - lax.shift_right_logical(i, 1)
            y = plsc.bitcast(i, jnp.float32)
            half = var * 0.5
            y = y * (1.5 - half * y * y)
            y = y * (1.5 - half * y * y)
            y = y * (1.5 - half * y * y)
            for c in range(HC):
                a = gam[c] * y
                rows_v[t, pl.ds(c * L, L)] = (xs[c] - mean) * a + bet[c]
            return tcarry

        lax.fori_loop(0, K, tok_body, 0)
        pltpu.sync_copy(rows_v, out_hbm.at[pl.ds(base + j * K, K)])
        return carry

    lax.fori_loop(0, CH, chunk_body, 0)


def kernel(input_ids, word_embeddings, position_embeddings,
           token_type_embeddings, ln_gamma, ln_beta):
    b, s = input_ids.shape
    _, h = word_embeddings.shape
    ids3 = input_ids.reshape(NW, CH, K)
    pe = position_embeddings[:s]
    tte0 = token_type_embeddings[0]
    out = _embed_ln(ids3, pe, tte0, ln_gamma, ln_beta, word_embeddings)
    return out.reshape(b, s, h)


# speedup vs baseline: 4.3853x; 1.5094x over previous
import functools

import jax
import jax.numpy as jnp
from jax import lax
from jax.experimental import pallas as pl
from jax.experimental.pallas import tpu as pltpu
from jax.experimental.pallas import tpu_sc as plsc

NC = 2
NS = 16
NW = NC * NS
L = 16

VOCAB = 100000
HID = 128
B = 1024
S = 200
HC = HID // L
K = 64
TOK_PER_W = B * S // NW
CH = TOK_PER_W // K
NB = 2
SB = 256
EPS = 1e-12

_MESH = plsc.VectorSubcoreMesh(
    core_axis_name="c", subcore_axis_name="s", num_cores=NC, num_subcores=NS
)


@functools.partial(
    pl.kernel,
    mesh=_MESH,
    compiler_params=pltpu.CompilerParams(needs_layout_passes=False),
    out_type=jax.ShapeDtypeStruct((B * S, HID), jnp.float32),
    scratch_types=[
        pltpu.VMEM((CH, K), jnp.int32),
        pltpu.VMEM((K, HID), jnp.float32),
        pltpu.VMEM((K, HID), jnp.float32),
        pltpu.VMEM((K, HID), jnp.float32),
        pltpu.VMEM((K, HID), jnp.float32),
        pltpu.VMEM((SB, HID), jnp.float32),
        pltpu.VMEM((HID,), jnp.float32),
        pltpu.VMEM((HID,), jnp.float32),
        pltpu.VMEM((HID,), jnp.float32),
        pltpu.SemaphoreType.DMA,
        pltpu.SemaphoreType.DMA,
        pltpu.SemaphoreType.DMA,
        pltpu.SemaphoreType.DMA,
    ],
)
def _embed_ln(ids_hbm, pe_hbm, tte_hbm, gam_hbm, bet_hbm, table_hbm, out_hbm,
              idx_v, rows0, rows1, outb0, outb1, bias_v, tte_v, gam_v, bet_v,
              gsem0, gsem1, ssem0, ssem1):
    wid = lax.axis_index("s") * NC + lax.axis_index("c")

    pltpu.sync_copy(pe_hbm, bias_v.at[pl.ds(0, S)])
    pltpu.sync_copy(pe_hbm.at[pl.ds(0, SB - S)], bias_v.at[pl.ds(S, SB - S)])
    pltpu.sync_copy(tte_hbm, tte_v)
    pltpu.sync_copy(gam_hbm, gam_v)
    pltpu.sync_copy(bet_hbm, bet_v)
    pltpu.sync_copy(ids_hbm.at[wid], idx_v)

    ttec = [tte_v[pl.ds(c * L, L)] for c in range(HC)]

    def bias_body(t, carry):
        for c in range(HC):
            bias_v[t, pl.ds(c * L, L)] = bias_v[t, pl.ds(c * L, L)] + ttec[c]
        return carry

    lax.fori_loop(0, SB, bias_body, 0)

    gam = [gam_v[pl.ds(c * L, L)] for c in range(HC)]
    bet = [bet_v[pl.ds(c * L, L)] for c in range(HC)]
    base = wid * TOK_PER_W

    lane = lax.iota(jnp.int32, L)
    dnums = lax.GatherDimensionNumbers(
        offset_dims=(), collapsed_slice_dims=(0,), start_index_map=(0,))

    def xlane_sum(v):
        for sh in (8, 4, 2, 1):
            perm = lax.bitwise_xor(lane, jnp.int32(sh))
            swapped = lax.gather(
                v, perm[:, None], dnums, slice_sizes=(1,),
                mode=lax.GatherScatterMode.PROMISE_IN_BOUNDS)
            v = v + swapped
        return v

    bufs = [(rows0, outb0, gsem0, ssem0), (rows1, outb1, gsem1, ssem1)]

    for b in range(NB):
        pltpu.async_copy(table_hbm.at[idx_v.at[b]], bufs[b][0], bufs[b][2])

    def pair_body(jj, carry):
        for b in range(NB):
            j = jj * NB + b
            rows_b, outb_b, gsem_b, ssem_b = bufs[b]
            pltpu.make_async_copy(
                table_hbm.at[pl.ds(0, K)], rows_b, gsem_b).wait()

            @pl.when(jj > 0)
            def _():
                pltpu.make_async_copy(
                    outb_b, out_hbm.at[pl.ds(0, K)], ssem_b).wait()

            poff = lax.rem(j * K, S)

            def tok_body(t, tcarry):
                xs = []
                for c in range(HC):
                    xs.append(rows_b[t, pl.ds(c * L, L)]
                              + bias_v[poff + t, pl.ds(c * L, L)])
                s1 = xs[0]
                s2 = xs[0] * xs[0]
                for c in range(1, HC):
                    s1 = s1 + xs[c]
                    s2 = s2 + xs[c] * xs[c]
                tot = xlane_sum(s1)
                tot2 = xlane_sum(s2)
                mean = tot * (1.0 / HID)
                var = tot2 * (1.0 / HID) - mean * mean + EPS
                i = plsc.bitcast(var, jnp.int32)
                i = jnp.int32(0x5F3759DF) - lax.shift_right_logical(i, 1)
                y = plsc.bitcast(i, jnp.float32)
                half = var * 0.5
                y = y * (1.5 - half * y * y)
                y = y * (1.5 - half * y * y)
                y = y * (1.5 - half * y * y)
                for c in range(HC):
                    a = gam[c] * y
                    outb_b[t, pl.ds(c * L, L)] = (xs[c] - mean) * a + bet[c]
                return tcarry

            lax.fori_loop(0, K, tok_body, 0)

            @pl.when(j + NB < CH)
            def _():
                pltpu.async_copy(
                    table_hbm.at[idx_v.at[j + NB]], rows_b, gsem_b)

            pltpu.async_copy(
                outb_b, out_hbm.at[pl.ds(base + j * K, K)], ssem_b)
        return carry

    lax.fori_loop(0, CH // NB, pair_body, 0)

    for b in range(NB):
        pltpu.make_async_copy(
            bufs[b][1], out_hbm.at[pl.ds(0, K)], bufs[b][3]).wait()


def kernel(input_ids, word_embeddings, position_embeddings,
           token_type_embeddings, ln_gamma, ln_beta):
    b, s = input_ids.shape
    _, h = word_embeddings.shape
    ids3 = input_ids.reshape(NW, CH, K)
    pe = position_embeddings[:s]
    tte0 = token_type_embeddings[0]
    out = _embed_ln(ids3, pe, tte0, ln_gamma, ln_beta, word_embeddings)
    return out.reshape(b, s, h)


# speedup vs baseline: 4.9265x; 1.1234x over previous
import functools

import jax
import jax.numpy as jnp
from jax import lax
from jax.experimental import pallas as pl
from jax.experimental.pallas import tpu as pltpu
from jax.experimental.pallas import tpu_sc as plsc

NC = 2
NS = 16
NW = NC * NS
L = 16

VOCAB = 100000
HID = 128
B = 1024
S = 200
HC = HID // L
K = 64
TOK_PER_W = B * S // NW
CH = TOK_PER_W // K
NB = 2
UNROLL = 4
SB = 256
EPS = 1e-12

_MESH = plsc.VectorSubcoreMesh(
    core_axis_name="c", subcore_axis_name="s", num_cores=NC, num_subcores=NS
)


@functools.partial(
    pl.kernel,
    mesh=_MESH,
    compiler_params=pltpu.CompilerParams(needs_layout_passes=False),
    out_type=jax.ShapeDtypeStruct((B * S, HID), jnp.float32),
    scratch_types=[
        pltpu.VMEM((CH, K), jnp.int32),
        pltpu.VMEM((K, HID), jnp.float32),
        pltpu.VMEM((K, HID), jnp.float32),
        pltpu.VMEM((K, HID), jnp.float32),
        pltpu.VMEM((K, HID), jnp.float32),
        pltpu.VMEM((SB, HID), jnp.float32),
        pltpu.VMEM((HID,), jnp.float32),
        pltpu.VMEM((HID,), jnp.float32),
        pltpu.VMEM((HID,), jnp.float32),
        pltpu.SemaphoreType.DMA,
        pltpu.SemaphoreType.DMA,
        pltpu.SemaphoreType.DMA,
        pltpu.SemaphoreType.DMA,
    ],
)
def _embed_ln(ids_hbm, pe_hbm, tte_hbm, gam_hbm, bet_hbm, table_hbm, out_hbm,
              idx_v, rows0, rows1, outb0, outb1, bias_v, tte_v, gam_v, bet_v,
              gsem0, gsem1, ssem0, ssem1):
    wid = lax.axis_index("s") * NC + lax.axis_index("c")

    pltpu.sync_copy(pe_hbm, bias_v.at[pl.ds(0, S)])
    pltpu.sync_copy(pe_hbm.at[pl.ds(0, SB - S)], bias_v.at[pl.ds(S, SB - S)])
    pltpu.sync_copy(tte_hbm, tte_v)
    pltpu.sync_copy(gam_hbm, gam_v)
    pltpu.sync_copy(bet_hbm, bet_v)
    pltpu.sync_copy(ids_hbm.at[wid], idx_v)

    ttec = [tte_v[pl.ds(c * L, L)] for c in range(HC)]

    def bias_body(t, carry):
        for c in range(HC):
            bias_v[t, pl.ds(c * L, L)] = bias_v[t, pl.ds(c * L, L)] + ttec[c]
        return carry

    lax.fori_loop(0, SB, bias_body, 0)

    gam = [gam_v[pl.ds(c * L, L)] for c in range(HC)]
    bet = [bet_v[pl.ds(c * L, L)] for c in range(HC)]
    base = wid * TOK_PER_W

    lane = lax.iota(jnp.int32, L)
    dnums = lax.GatherDimensionNumbers(
        offset_dims=(), collapsed_slice_dims=(0,), start_index_map=(0,))

    def xlane_sum(v):
        for sh in (8, 4, 2, 1):
            perm = lax.bitwise_xor(lane, jnp.int32(sh))
            swapped = lax.gather(
                v, perm[:, None], dnums, slice_sizes=(1,),
                mode=lax.GatherScatterMode.PROMISE_IN_BOUNDS)
            v = v + swapped
        return v

    bufs = [(rows0, outb0, gsem0, ssem0), (rows1, outb1, gsem1, ssem1)]

    for b in range(NB):
        pltpu.async_copy(table_hbm.at[idx_v.at[b]], bufs[b][0], bufs[b][2])

    def pair_body(jj, carry):
        for b in range(NB):
            j = jj * NB + b
            rows_b, outb_b, gsem_b, ssem_b = bufs[b]
            pltpu.make_async_copy(
                table_hbm.at[pl.ds(0, K)], rows_b, gsem_b).wait()

            @pl.when(jj > 0)
            def _():
                pltpu.make_async_copy(
                    outb_b, out_hbm.at[pl.ds(0, K)], ssem_b).wait()

            poff = lax.rem(j * K, S)

            def ln_token(t):
                xs = []
                for c in range(HC):
                    xs.append(rows_b[t, pl.ds(c * L, L)]
                              + bias_v[poff + t, pl.ds(c * L, L)])
                s1 = xs[0]
                s2 = xs[0] * xs[0]
                for c in range(1, HC):
                    s1 = s1 + xs[c]
                    s2 = s2 + xs[c] * xs[c]
                tot = xlane_sum(s1)
                tot2 = xlane_sum(s2)
                mean = tot * (1.0 / HID)
                var = tot2 * (1.0 / HID) - mean * mean + EPS
                i = plsc.bitcast(var, jnp.int32)
                i = jnp.int32(0x5F3759DF) - lax.shift_right_logical(i, 1)
                y = plsc.bitcast(i, jnp.float32)
                half = var * 0.5
                y = y * (1.5 - half * y * y)
                y = y * (1.5 - half * y * y)
                for c in range(HC):
                    a = gam[c] * y
                    outb_b[t, pl.ds(c * L, L)] = (xs[c] - mean) * a + bet[c]

            def tok_body(t, tcarry):
                t0 = t * UNROLL
                for u in range(UNROLL):
                    ln_token(t0 + u)
                return tcarry

            lax.fori_loop(0, K // UNROLL, tok_body, 0)

            @pl.when(j + NB < CH)
            def _():
                pltpu.async_copy(
                    table_hbm.at[idx_v.at[j + NB]], rows_b, gsem_b)

            pltpu.async_copy(
                outb_b, out_hbm.at[pl.ds(base + j * K, K)], ssem_b)
        return carry

    lax.fori_loop(0, CH // NB, pair_body, 0)

    for b in range(NB):
        pltpu.make_async_copy(
            bufs[b][1], out_hbm.at[pl.ds(0, K)], bufs[b][3]).wait()


def kernel(input_ids, word_embeddings, position_embeddings,
           token_type_embeddings, ln_gamma, ln_beta):
    b, s = input_ids.shape
    _, h = word_embeddings.shape
    ids3 = input_ids.reshape(NW, CH, K)
    pe = position_embeddings[:s]
    tte0 = token_type_embeddings[0]
    out = _embed_ln(ids3, pe, tte0, ln_gamma, ln_beta, word_embeddings)
    return out.reshape(b, s, h)


# speedup vs baseline: 8.9970x; 1.8263x over previous
import functools

import jax
import jax.numpy as jnp
from jax import lax
from jax.experimental import pallas as pl
from jax.experimental.pallas import tpu as pltpu
from jax.experimental.pallas import tpu_sc as plsc

NC = 2
NS = 16
NW = NC * NS
L = 16

VOCAB = 100000
HID = 128
B = 1024
S = 200
HC = HID // L
K = 64
TOK_PER_W = B * S // NW
CH = TOK_PER_W // K
NB = 2
UNROLL = 4
SB = 256
EPS = 1e-12

_MESH = plsc.VectorSubcoreMesh(
    core_axis_name="c", subcore_axis_name="s", num_cores=NC, num_subcores=NS
)


@functools.partial(
    pl.kernel,
    mesh=_MESH,
    compiler_params=pltpu.CompilerParams(needs_layout_passes=False),
    out_type=jax.ShapeDtypeStruct((B * S, HID), jnp.float32),
    scratch_types=[
        pltpu.VMEM((CH, K), jnp.int32),
        pltpu.VMEM((K, HID), jnp.float32),
        pltpu.VMEM((K, HID), jnp.float32),
        pltpu.VMEM((K, HID), jnp.float32),
        pltpu.VMEM((K, HID), jnp.float32),
        pltpu.VMEM((SB, HID), jnp.float32),
        pltpu.VMEM((HID,), jnp.float32),
        pltpu.VMEM((HID,), jnp.float32),
        pltpu.VMEM((HID,), jnp.float32),
        pltpu.SemaphoreType.DMA,
        pltpu.SemaphoreType.DMA,
        pltpu.SemaphoreType.DMA,
        pltpu.SemaphoreType.DMA,
    ],
)
def _embed_ln(ids_hbm, pe_hbm, tte_hbm, gam_hbm, bet_hbm, table_hbm, out_hbm,
              idx_v, rows0, rows1, outb0, outb1, bias_v, tte_v, gam_v, bet_v,
              gsem0, gsem1, ssem0, ssem1):
    wid = lax.axis_index("s") * NC + lax.axis_index("c")

    pltpu.sync_copy(pe_hbm, bias_v.at[pl.ds(0, S)])
    pltpu.sync_copy(pe_hbm.at[pl.ds(0, SB - S)], bias_v.at[pl.ds(S, SB - S)])
    pltpu.sync_copy(tte_hbm, tte_v)
    pltpu.sync_copy(gam_hbm, gam_v)
    pltpu.sync_copy(bet_hbm, bet_v)
    pltpu.sync_copy(ids_hbm.at[wid], idx_v)

    ttec = [tte_v[pl.ds(c * L, L)] for c in range(HC)]

    def bias_body(t, carry):
        for c in range(HC):
            bias_v[t, pl.ds(c * L, L)] = bias_v[t, pl.ds(c * L, L)] + ttec[c]
        return carry

    lax.fori_loop(0, SB, bias_body, 0)

    gam = [gam_v[pl.ds(c * L, L)] for c in range(HC)]
    bet = [bet_v[pl.ds(c * L, L)] for c in range(HC)]
    base = wid * TOK_PER_W

    lane = lax.iota(jnp.int32, L)
    dnums = lax.GatherDimensionNumbers(
        offset_dims=(), collapsed_slice_dims=(0,), start_index_map=(0,))

    def xlane_sum(v):
        for sh in (8, 4, 2, 1):
            perm = lax.bitwise_xor(lane, jnp.int32(sh))
            swapped = lax.gather(
                v, perm[:, None], dnums, slice_sizes=(1,),
                mode=lax.GatherScatterMode.PROMISE_IN_BOUNDS)
            v = v + swapped
        return v

    bufs = [(rows0, outb0, gsem0, ssem0), (rows1, outb1, gsem1, ssem1)]

    for b in range(NB):
        pltpu.async_copy(table_hbm.at[idx_v.at[b]], bufs[b][0], bufs[b][2])

    def pair_body(jj, carry):
        for b in range(NB):
            j = jj * NB + b
            rows_b, outb_b, gsem_b, ssem_b = bufs[b]
            pltpu.make_async_copy(
                table_hbm.at[pl.ds(0, K)], rows_b, gsem_b).wait()

            @pl.when(jj > 0)
            def _():
                pltpu.make_async_copy(
                    outb_b, out_hbm.at[pl.ds(0, K)], ssem_b).wait()

            poff = lax.rem(j * K, S)

            def ln_token(t):
                xs = []
                for c in range(HC):
                    xs.append(rows_b[t, pl.ds(c * L, L)]
                              + bias_v[poff + t, pl.ds(c * L, L)])
                s1 = xs[0]
                s2 = xs[0] * xs[0]
                for c in range(1, HC):
                    s1 = s1 + xs[c]
                    s2 = s2 + xs[c] * xs[c]
                tot = xlane_sum(s1)
                tot2 = xlane_sum(s2)
                mean = tot * (1.0 / HID)
                var = tot2 * (1.0 / HID) - mean * mean + EPS
                i = plsc.bitcast(var, jnp.int32)
                i = jnp.int32(0x5F3759DF) - lax.shift_right_logical(i, 1)
                y = plsc.bitcast(i, jnp.float32)
                half = var * 0.5
                y = y * (1.5 - half * y * y)
                y = y * (1.5 - half * y * y)
                for c in range(HC):
                    a = gam[c] * y
                    outb_b[t, pl.ds(c * L, L)] = (xs[c] - mean) * a + bet[c]

            @plsc.parallel_loop(0, K, 1, unroll=UNROLL)
            def _(t):
                ln_token(t)

            @pl.when(j + NB < CH)
            def _():
                pltpu.async_copy(
                    table_hbm.at[idx_v.at[j + NB]], rows_b, gsem_b)

            pltpu.async_copy(
                outb_b, out_hbm.at[pl.ds(base + j * K, K)], ssem_b)
        return carry

    lax.fori_loop(0, CH // NB, pair_body, 0)

    for b in range(NB):
        pltpu.make_async_copy(
            bufs[b][1], out_hbm.at[pl.ds(0, K)], bufs[b][3]).wait()


def kernel(input_ids, word_embeddings, position_embeddings,
           token_type_embeddings, ln_gamma, ln_beta):
    b, s = input_ids.shape
    _, h = word_embeddings.shape
    ids3 = input_ids.reshape(NW, CH, K)
    pe = position_embeddings[:s]
    tte0 = token_type_embeddings[0]
    out = _embed_ln(ids3, pe, tte0, ln_gamma, ln_beta, word_embeddings)
    return out.reshape(b, s, h)
